# Initial kernel scaffold; baseline (speedup 1.0000x reference)
import jax
import jax.numpy as jnp
from jax.experimental import pallas as pl


def kernel(x1, x2, edge_index_v1, edge_index_v2, W1, b1, W2, b2, prelu_a, bn1_g, bn1_b, lin1_W, lin1_b, prelu2_a, bn2_g, bn2_b, lin2_W, lin2_b):
    raise NotImplementedError("write your pallas kernel here")


# speedup vs baseline: 7.4550x; 7.4550x over previous
import functools

import jax
import jax.numpy as jnp
from jax import lax
from jax.experimental import pallas as pl
from jax.experimental.pallas import tpu as pltpu
from jax.experimental.pallas import tpu_sc as plsc

_N = 10000
_E = 320000
_D = 128
_PH = 512
_EPS_BN = 1e-5

_NC = 2
_NS = 16
_CH = 128
_EPT = _E // _NS
_IB = 8
_OB = 20
_CPT = _OB * _IB
_PAD = _CPT * _CH - _EPT
_ACC_N = _N + 8

_NB = 20
_RB = (2 * _N) // _NB
_NBV = _NB // 2

_SPLIT = 632


def _per_tile_copy(s, total_rows, copy_fn):
    start = s * _SPLIT

    @pl.when(s < _NS - 1)
    def _main():
        copy_fn(start, _SPLIT)

    @pl.when(s == _NS - 1)
    def _tail():
        copy_fn(start, total_rows - (_NS - 1) * _SPLIT)

_sc_mesh = plsc.VectorSubcoreMesh(core_axis_name="c", subcore_axis_name="s")


def _hist_body(dst_hbm, ones_hbm, out_hbm, acc, ones_v, dst_v):
    c = lax.axis_index("c")
    s = lax.axis_index("s")

    def init(start, n):
        pltpu.sync_copy(ones_hbm.at[pl.ds(0, n)], acc.at[pl.ds(start, n)])

    _per_tile_copy(s, _ACC_N, init)
    pltpu.sync_copy(ones_hbm.at[pl.ds(0, _CH)], ones_v)
    plsc.subcore_barrier()

    def outer(o, carry):
        pltpu.sync_copy(dst_hbm.at[c, s, pl.ds(o * _IB, _IB)], dst_v)

        def inner(k, cc):
            pltpu.sync_copy(ones_v, acc.at[dst_v.at[k]], add=True)
            return cc

        return lax.fori_loop(0, _IB, inner, carry)

    lax.fori_loop(0, _OB, outer, 0)
    plsc.subcore_barrier()

    def writeback(start, n):
        pltpu.sync_copy(acc.at[pl.ds(start, n)],
                        out_hbm.at[c].at[pl.ds(start, n)])

    _per_tile_copy(s, _N, writeback)


def _degree_hist(dsts, ones):
    return pl.kernel(
        _hist_body,
        out_type=jax.ShapeDtypeStruct((_NC, _N, 8), jnp.float32),
        mesh=_sc_mesh,
        scratch_types=[
            pltpu.VMEM_SHARED((_ACC_N, 8), jnp.float32),
            pltpu.VMEM((_CH, 8), jnp.float32),
            pltpu.VMEM((_IB, _CH), jnp.int32),
        ],
    )(dsts, ones)


def _agg_body(hp_hbm, src_hbm, dst_hbm, zeros_hbm, out_hbm,
              acc, src_v, dst_v, rows_v, sem):
    c = lax.axis_index("c")
    s = lax.axis_index("s")

    def init(start, n):
        pltpu.sync_copy(zeros_hbm.at[pl.ds(0, n)], acc.at[pl.ds(start, n)])

    _per_tile_copy(s, _ACC_N, init)
    plsc.subcore_barrier()

    def outer(o, carry):
        pltpu.sync_copy(src_hbm.at[c, s, pl.ds(o * _IB, _IB)], src_v)
        pltpu.sync_copy(dst_hbm.at[c, s, pl.ds(o * _IB, _IB)], dst_v)

        def inner(k, cc):
            pltpu.async_copy(hp_hbm.at[src_v.at[k]], rows_v, sem).wait()
            pltpu.sync_copy(rows_v, acc.at[dst_v.at[k]], add=True)
            return cc

        return lax.fori_loop(0, _IB, inner, carry)

    lax.fori_loop(0, _OB, outer, 0)
    plsc.subcore_barrier()

    def writeback(start, n):
        pltpu.sync_copy(acc.at[pl.ds(start, n)],
                        out_hbm.at[pl.ds(c * _N + start, n)])

    _per_tile_copy(s, _N, writeback)


def _aggregate(hp, srcs, dsts, zeros):
    return pl.kernel(
        _agg_body,
        out_type=jax.ShapeDtypeStruct((2 * _N, _D), jnp.float32),
        mesh=_sc_mesh,
        scratch_types=[
            pltpu.VMEM_SHARED((_ACC_N, _D), jnp.float32),
            pltpu.VMEM((_IB, _CH), jnp.int32),
            pltpu.VMEM((_IB, _CH), jnp.int32),
            pltpu.VMEM((_CH, _D), jnp.float32),
            pltpu.SemaphoreType.DMA,
        ],
    )(hp, srcs, dsts, zeros)


def _mm_scale_body(deg_ref, x_ref, w_ref, h_ref, hp_ref):
    dinv = lax.rsqrt(deg_ref[:, 0:1])
    h = jnp.dot(x_ref[...], w_ref[...], preferred_element_type=jnp.float32)
    h_ref[...] = h
    hp_ref[...] = h * dinv


def _mm_scale(deg, x, w):
    return pl.pallas_call(
        _mm_scale_body,
        grid=(_NB,),
        in_specs=[
            pl.BlockSpec((_RB, 8), lambda j: (j, 0)),
            pl.BlockSpec((_RB, _D), lambda j: (j, 0)),
            pl.BlockSpec((_D, _D), lambda j: (0, 0)),
        ],
        out_specs=[
            pl.BlockSpec((_RB, _D), lambda j: (j, 0)),
            pl.BlockSpec((_RB, _D), lambda j: (j, 0)),
        ],
        out_shape=[
            jax.ShapeDtypeStruct((2 * _N, _D), jnp.float32),
            jax.ShapeDtypeStruct((2 * _N, _D), jnp.float32),
        ],
    )(deg, x, w)


def _layer_mm_body(deg_ref, agg_ref, h_ref, b_ref, a_ref, w_ref,
                   h2_ref, hp2_ref):
    dinv = lax.rsqrt(deg_ref[:, 0:1])
    a = a_ref[0, 0]
    t = dinv * agg_ref[...] + (dinv * dinv) * h_ref[...] + b_ref[...]
    xn = jnp.where(t >= 0, t, a * t)
    h2 = jnp.dot(xn, w_ref[...], preferred_element_type=jnp.float32)
    h2_ref[...] = h2
    hp2_ref[...] = h2 * dinv


def _layer_mm(deg, agg, h, b, a, w):
    return pl.pallas_call(
        _layer_mm_body,
        grid=(_NB,),
        in_specs=[
            pl.BlockSpec((_RB, 8), lambda j: (j, 0)),
            pl.BlockSpec((_RB, _D), lambda j: (j, 0)),
            pl.BlockSpec((_RB, _D), lambda j: (j, 0)),
            pl.BlockSpec((1, _D), lambda j: (0, 0)),
            pl.BlockSpec((1, 1), lambda j: (0, 0)),
            pl.BlockSpec((_D, _D), lambda j: (0, 0)),
        ],
        out_specs=[
            pl.BlockSpec((_RB, _D), lambda j: (j, 0)),
            pl.BlockSpec((_RB, _D), lambda j: (j, 0)),
        ],
        out_shape=[
            jax.ShapeDtypeStruct((2 * _N, _D), jnp.float32),
            jax.ShapeDtypeStruct((2 * _N, _D), jnp.float32),
        ],
    )(deg, agg, h, b, a, w)


def _final_layer_body(deg_ref, agg_ref, h_ref, b_ref, a_ref,
                      vs_ref, s1_ref, s2_ref):
    dinv = lax.rsqrt(deg_ref[:, 0:1])
    a = a_ref[0, 0]
    t = dinv * agg_ref[...] + (dinv * dinv) * h_ref[...] + b_ref[...]
    v = jnp.where(t >= 0, t, a * t)
    vs_ref[...] = v
    cs = jnp.broadcast_to(jnp.sum(v, axis=0, keepdims=True), (8, _D))
    cq = jnp.broadcast_to(jnp.sum(v * v, axis=0, keepdims=True), (8, _D))
    j = pl.program_id(0)

    @pl.when(j % _NBV == 0)
    def _init():
        s1_ref[...] = cs
        s2_ref[...] = cq

    @pl.when(j % _NBV != 0)
    def _accum():
        s1_ref[...] = s1_ref[...] + cs
        s2_ref[...] = s2_ref[...] + cq


def _final_layer(deg, agg, h, b, a):
    return pl.pallas_call(
        _final_layer_body,
        grid=(_NB,),
        in_specs=[
            pl.BlockSpec((_RB, 8), lambda j: (j, 0)),
            pl.BlockSpec((_RB, _D), lambda j: (j, 0)),
            pl.BlockSpec((_RB, _D), lambda j: (j, 0)),
            pl.BlockSpec((1, _D), lambda j: (0, 0)),
            pl.BlockSpec((1, 1), lambda j: (0, 0)),
        ],
        out_specs=[
            pl.BlockSpec((_RB, _D), lambda j: (j, 0)),
            pl.BlockSpec((8, _D), lambda j: (j // _NBV, 0)),
            pl.BlockSpec((8, _D), lambda j: (j // _NBV, 0)),
        ],
        out_shape=[
            jax.ShapeDtypeStruct((2 * _N, _D), jnp.float32),
            jax.ShapeDtypeStruct((16, _D), jnp.float32),
            jax.ShapeDtypeStruct((16, _D), jnp.float32),
        ],
    )(deg, agg, h, b, a)


def _pred1_body(vs_ref, s1_ref, s2_ref, g_ref, bb_ref, w_ref, lb_ref, a_ref,
                z_ref, t1_ref, t2_ref):
    inv_n = 1.0 / _N
    mu = s1_ref[0:1, :] * inv_n
    var = s2_ref[0:1, :] * inv_n - mu * mu
    xn = (vs_ref[...] - mu) * lax.rsqrt(var + _EPS_BN) * g_ref[...] + bb_ref[...]
    z0 = jnp.dot(xn, w_ref[...], preferred_element_type=jnp.float32) + lb_ref[...]
    a = a_ref[0, 0]
    z = jnp.where(z0 >= 0, z0, a * z0)
    z_ref[...] = z
    cs = jnp.broadcast_to(jnp.sum(z, axis=0, keepdims=True), (8, _PH))
    cq = jnp.broadcast_to(jnp.sum(z * z, axis=0, keepdims=True), (8, _PH))
    j = pl.program_id(0)

    @pl.when(j % _NBV == 0)
    def _init():
        t1_ref[...] = cs
        t2_ref[...] = cq

    @pl.when(j % _NBV != 0)
    def _accum():
        t1_ref[...] = t1_ref[...] + cs
        t2_ref[...] = t2_ref[...] + cq


def _pred1(vs, s1, s2, g, bb, w, lb, a):
    return pl.pallas_call(
        _pred1_body,
        grid=(_NB,),
        in_specs=[
            pl.BlockSpec((_RB, _D), lambda j: (j, 0)),
            pl.BlockSpec((8, _D), lambda j: (j // _NBV, 0)),
            pl.BlockSpec((8, _D), lambda j: (j // _NBV, 0)),
            pl.BlockSpec((1, _D), lambda j: (0, 0)),
            pl.BlockSpec((1, _D), lambda j: (0, 0)),
            pl.BlockSpec((_D, _PH), lambda j: (0, 0)),
            pl.BlockSpec((1, _PH), lambda j: (0, 0)),
            pl.BlockSpec((1, 1), lambda j: (0, 0)),
        ],
        out_specs=[
            pl.BlockSpec((_RB, _PH), lambda j: (j, 0)),
            pl.BlockSpec((8, _PH), lambda j: (j // _NBV, 0)),
            pl.BlockSpec((8, _PH), lambda j: (j // _NBV, 0)),
        ],
        out_shape=[
            jax.ShapeDtypeStruct((2 * _N, _PH), jnp.float32),
            jax.ShapeDtypeStruct((16, _PH), jnp.float32),
            jax.ShapeDtypeStruct((16, _PH), jnp.float32),
        ],
    )(vs, s1, s2, g, bb, w, lb, a)


def _loss_body(z1_ref, z2_ref, t1_ref, t2_ref, g_ref, bb_ref, w_ref, lb_ref,
               vs1_ref, vs2_ref, loss_ref):
    inv_n = 1.0 / _N

    def bn(z, k):
        mu = t1_ref[8 * k:8 * k + 1, :] * inv_n
        var = t2_ref[8 * k:8 * k + 1, :] * inv_n - mu * mu
        return (z - mu) * lax.rsqrt(var + _EPS_BN) * g_ref[...] + bb_ref[...]

    p1 = jnp.dot(bn(z1_ref[...], 0), w_ref[...],
                 preferred_element_type=jnp.float32) + lb_ref[...]
    p2 = jnp.dot(bn(z2_ref[...], 1), w_ref[...],
                 preferred_element_type=jnp.float32) + lb_ref[...]

    def nrm(x):
        n = jnp.sqrt(jnp.sum(x * x, axis=1, keepdims=True))
        return x / jnp.maximum(n, 1e-12)

    c1 = jnp.sum(nrm(p1) * nrm(vs2_ref[...]), axis=1)
    c2 = jnp.sum(nrm(p2) * nrm(vs1_ref[...]), axis=1)
    part = jnp.reshape(jnp.sum(4.0 - 2.0 * c1 - 2.0 * c2) * inv_n, (1, 1))
    j = pl.program_id(0)

    @pl.when(j == 0)
    def _init():
        loss_ref[...] = part

    @pl.when(j != 0)
    def _accum():
        loss_ref[...] = loss_ref[...] + part


def _loss(z, t1, t2, g, bb, w, lb, vs):
    return pl.pallas_call(
        _loss_body,
        grid=(_NBV,),
        in_specs=[
            pl.BlockSpec((_RB, _PH), lambda j: (j, 0)),
            pl.BlockSpec((_RB, _PH), lambda j: (j + _NBV, 0)),
            pl.BlockSpec((16, _PH), lambda j: (0, 0)),
            pl.BlockSpec((16, _PH), lambda j: (0, 0)),
            pl.BlockSpec((1, _PH), lambda j: (0, 0)),
            pl.BlockSpec((1, _PH), lambda j: (0, 0)),
            pl.BlockSpec((_PH, _D), lambda j: (0, 0)),
            pl.BlockSpec((1, _D), lambda j: (0, 0)),
            pl.BlockSpec((_RB, _D), lambda j: (j, 0)),
            pl.BlockSpec((_RB, _D), lambda j: (j + _NBV, 0)),
        ],
        out_specs=pl.BlockSpec((1, 1), lambda j: (0, 0)),
        out_shape=jax.ShapeDtypeStruct((1, 1), jnp.float32),
    )(z, z, t1, t2, g, bb, w, lb, vs, vs)


def kernel(x1, x2, edge_index_v1, edge_index_v2, W1, b1, W2, b2, prelu_a,
           bn1_g, bn1_b, lin1_W, lin1_b, prelu2_a, bn2_g, bn2_b, lin2_W,
           lin2_b):
    f32 = jnp.float32

    def prep_idx(col, fill):
        a = col.astype(jnp.int32).reshape(_NS, _EPT)
        pad = jnp.full((_NS, _PAD), fill, jnp.int32)
        return jnp.concatenate([a, pad], axis=1).reshape(_NS, _CPT, _CH)

    src = jnp.stack([
        prep_idx(edge_index_v1[0], 0),
        prep_idx(edge_index_v2[0].astype(jnp.int32) + _N, 0),
    ])
    dst = jnp.stack([
        prep_idx(edge_index_v1[1], _N),
        prep_idx(edge_index_v2[1], _N),
    ])
    ones8 = jnp.ones((_SPLIT, 8), f32)
    zeros = jnp.zeros((_SPLIT, _D), f32)
    xcat = jnp.concatenate([x1, x2], axis=0)

    a1 = jnp.reshape(prelu_a.astype(f32), (1, 1))
    a2 = jnp.reshape(prelu2_a.astype(f32), (1, 1))
    b1r = jnp.reshape(b1, (1, _D))
    b2r = jnp.reshape(b2, (1, _D))
    g1r = jnp.reshape(bn1_g, (1, _D))
    bb1r = jnp.reshape(bn1_b, (1, _D))
    lb1r = jnp.reshape(lin1_b, (1, _PH))
    g2r = jnp.reshape(bn2_g, (1, _PH))
    bb2r = jnp.reshape(bn2_b, (1, _PH))
    lb2r = jnp.reshape(lin2_b, (1, _D))

    deg = _degree_hist(dst, ones8).reshape(2 * _N, 8)

    h1, hp1 = _mm_scale(deg, xcat, W1)
    agg1 = _aggregate(hp1, src, dst, zeros)
    h2, hp2 = _layer_mm(deg, agg1, h1, b1r, a1, W2)
    agg2 = _aggregate(hp2, src, dst, zeros)
    vs, s1, s2 = _final_layer(deg, agg2, h2, b2r, a1)

    z, t1, t2 = _pred1(vs, s1, s2, g1r, bb1r, lin1_W, lb1r, a2)
    lossm = _loss(z, t1, t2, g2r, bb2r, lin2_W, lb2r, vs)

    return vs[:_N], vs[_N:], lossm[0, 0]


# speedup vs baseline: 8.6817x; 1.1646x over previous
import functools

import jax
import jax.numpy as jnp
from jax import lax
from jax.experimental import pallas as pl
from jax.experimental.pallas import tpu as pltpu
from jax.experimental.pallas import tpu_sc as plsc

_N = 10000
_E = 320000
_D = 128
_PH = 512
_EPS_BN = 1e-5

_NC = 2
_NS = 16
_CH = 128
_EPT = _E // _NS
_IB = 8
_OB = 20
_CPT = _OB * _IB
_PAD = _CPT * _CH - _EPT
_ACC_N = _N + 8

_NB = 20
_RB = (2 * _N) // _NB
_NBV = _NB // 2

_SPLIT = 632


def _per_tile_copy(s, total_rows, copy_fn):
    start = s * _SPLIT

    @pl.when(s < _NS - 1)
    def _main():
        copy_fn(start, _SPLIT)

    @pl.when(s == _NS - 1)
    def _tail():
        copy_fn(start, total_rows - (_NS - 1) * _SPLIT)

_sc_mesh = plsc.VectorSubcoreMesh(core_axis_name="c", subcore_axis_name="s")


def _hist_body(dst_hbm, ones_hbm, out_hbm, acc, ones_v, dst_v):
    c = lax.axis_index("c")
    s = lax.axis_index("s")

    def init(start, n):
        pltpu.sync_copy(ones_hbm.at[pl.ds(0, n)], acc.at[pl.ds(start, n)])

    _per_tile_copy(s, _ACC_N, init)
    pltpu.sync_copy(ones_hbm.at[pl.ds(0, _CH)], ones_v)
    plsc.subcore_barrier()

    def outer(o, carry):
        pltpu.sync_copy(dst_hbm.at[c, s, pl.ds(o * _IB, _IB)], dst_v)

        def inner(k, cc):
            pltpu.sync_copy(ones_v, acc.at[dst_v.at[k]], add=True)
            return cc

        return lax.fori_loop(0, _IB, inner, carry)

    lax.fori_loop(0, _OB, outer, 0)
    plsc.subcore_barrier()

    def writeback(start, n):
        pltpu.sync_copy(acc.at[pl.ds(start, n)],
                        out_hbm.at[c].at[pl.ds(start, n)])

    _per_tile_copy(s, _N, writeback)


def _degree_hist(dsts, ones):
    return pl.kernel(
        _hist_body,
        out_type=jax.ShapeDtypeStruct((_NC, _N, 8), jnp.float32),
        mesh=_sc_mesh,
        scratch_types=[
            pltpu.VMEM_SHARED((_ACC_N, 8), jnp.float32),
            pltpu.VMEM((_CH, 8), jnp.float32),
            pltpu.VMEM((_IB, _CH), jnp.int32),
        ],
    )(dsts, ones)


_NBUF = 2


def _agg_body(hp_hbm, src_hbm, dst_hbm, zeros_hbm, out_hbm,
              acc, src_v, dst_v, rows_v, gsem, ssem, isem):
    c = lax.axis_index("c")
    s = lax.axis_index("s")

    def init(start, n):
        pltpu.sync_copy(zeros_hbm.at[pl.ds(0, n)], acc.at[pl.ds(start, n)])

    _per_tile_copy(s, _ACC_N, init)
    pltpu.sync_copy(src_hbm.at[c, s, pl.ds(0, _IB)], src_v.at[0])
    pltpu.sync_copy(dst_hbm.at[c, s, pl.ds(0, _IB)], dst_v.at[0])
    plsc.subcore_barrier()

    for b in range(_NBUF):
        pltpu.async_copy(hp_hbm.at[src_v.at[0, b]], rows_v.at[b], gsem.at[b])

    def gwait(rb):
        pltpu.make_async_copy(hp_hbm.at[pl.ds(0, _CH)], rows_v.at[rb],
                              gsem.at[rb]).wait()

    def outer(oo, carry):
        for ib in range(2):
            o = 2 * oo + ib
            nslot = 1 - ib
            has_next = (oo < (_OB // 2) - 1) if ib else True

            def prefetch():
                pltpu.async_copy(
                    src_hbm.at[c, s, pl.ds((o + 1) * _IB, _IB)],
                    src_v.at[nslot], isem.at[0])
                pltpu.async_copy(
                    dst_hbm.at[c, s, pl.ds((o + 1) * _IB, _IB)],
                    dst_v.at[nslot], isem.at[1])

            def iwait():
                pltpu.make_async_copy(
                    src_hbm.at[c, s, pl.ds(0, _IB)], src_v.at[nslot],
                    isem.at[0]).wait()
                pltpu.make_async_copy(
                    dst_hbm.at[c, s, pl.ds(0, _IB)], dst_v.at[nslot],
                    isem.at[1]).wait()

            if ib == 0:
                prefetch()
            else:
                pl.when(has_next)(prefetch)

            for k in range(_IB):
                rb = k % _NBUF
                gwait(rb)
                pltpu.async_copy(rows_v.at[rb], acc.at[dst_v.at[ib, k]],
                                 ssem.at[rb], add=True).wait()

                kn = k + _NBUF
                if kn < _IB:
                    pltpu.async_copy(hp_hbm.at[src_v.at[ib, kn]],
                                     rows_v.at[rb], gsem.at[rb])
                else:
                    if kn == _IB:
                        if ib == 0:
                            iwait()
                        else:
                            pl.when(has_next)(iwait)

                    def gnext(kk=kn - _IB, rr=rb):
                        pltpu.async_copy(hp_hbm.at[src_v.at[nslot, kk]],
                                         rows_v.at[rr], gsem.at[rr])

                    if ib == 0:
                        gnext()
                    else:
                        pl.when(has_next)(gnext)
        return carry

    lax.fori_loop(0, _OB // 2, outer, 0)
    plsc.subcore_barrier()

    def writeback(start, n):
        pltpu.sync_copy(acc.at[pl.ds(start, n)],
                        out_hbm.at[pl.ds(c * _N + start, n)])

    _per_tile_copy(s, _N, writeback)


def _aggregate(hp, srcs, dsts, zeros):
    return pl.kernel(
        _agg_body,
        out_type=jax.ShapeDtypeStruct((2 * _N, _D), jnp.float32),
        mesh=_sc_mesh,
        scratch_types=[
            pltpu.VMEM_SHARED((_ACC_N, _D), jnp.float32),
            pltpu.VMEM((2, _IB, _CH), jnp.int32),
            pltpu.VMEM((2, _IB, _CH), jnp.int32),
            pltpu.VMEM((_NBUF, _CH, _D), jnp.float32),
            pltpu.SemaphoreType.DMA((_NBUF,)),
            pltpu.SemaphoreType.DMA((_NBUF,)),
            pltpu.SemaphoreType.DMA((2,)),
        ],
    )(hp, srcs, dsts, zeros)


def _mm_scale_body(deg_ref, x_ref, w_ref, h_ref, hp_ref):
    dinv = lax.rsqrt(deg_ref[:, 0:1])
    h = jnp.dot(x_ref[...], w_ref[...], preferred_element_type=jnp.float32)
    h_ref[...] = h
    hp_ref[...] = h * dinv


def _mm_scale(deg, x, w):
    return pl.pallas_call(
        _mm_scale_body,
        grid=(_NB,),
        in_specs=[
            pl.BlockSpec((_RB, 8), lambda j: (j, 0)),
            pl.BlockSpec((_RB, _D), lambda j: (j, 0)),
            pl.BlockSpec((_D, _D), lambda j: (0, 0)),
        ],
        out_specs=[
            pl.BlockSpec((_RB, _D), lambda j: (j, 0)),
            pl.BlockSpec((_RB, _D), lambda j: (j, 0)),
        ],
        out_shape=[
            jax.ShapeDtypeStruct((2 * _N, _D), jnp.float32),
            jax.ShapeDtypeStruct((2 * _N, _D), jnp.float32),
        ],
    )(deg, x, w)


def _layer_mm_body(deg_ref, agg_ref, h_ref, b_ref, a_ref, w_ref,
                   h2_ref, hp2_ref):
    dinv = lax.rsqrt(deg_ref[:, 0:1])
    a = a_ref[0, 0]
    t = dinv * agg_ref[...] + (dinv * dinv) * h_ref[...] + b_ref[...]
    xn = jnp.where(t >= 0, t, a * t)
    h2 = jnp.dot(xn, w_ref[...], preferred_element_type=jnp.float32)
    h2_ref[...] = h2
    hp2_ref[...] = h2 * dinv


def _layer_mm(deg, agg, h, b, a, w):
    return pl.pallas_call(
        _layer_mm_body,
        grid=(_NB,),
        in_specs=[
            pl.BlockSpec((_RB, 8), lambda j: (j, 0)),
            pl.BlockSpec((_RB, _D), lambda j: (j, 0)),
            pl.BlockSpec((_RB, _D), lambda j: (j, 0)),
            pl.BlockSpec((1, _D), lambda j: (0, 0)),
            pl.BlockSpec((1, 1), lambda j: (0, 0)),
            pl.BlockSpec((_D, _D), lambda j: (0, 0)),
        ],
        out_specs=[
            pl.BlockSpec((_RB, _D), lambda j: (j, 0)),
            pl.BlockSpec((_RB, _D), lambda j: (j, 0)),
        ],
        out_shape=[
            jax.ShapeDtypeStruct((2 * _N, _D), jnp.float32),
            jax.ShapeDtypeStruct((2 * _N, _D), jnp.float32),
        ],
    )(deg, agg, h, b, a, w)


def _final_layer_body(deg_ref, agg_ref, h_ref, b_ref, a_ref,
                      vs_ref, s1_ref, s2_ref):
    dinv = lax.rsqrt(deg_ref[:, 0:1])
    a = a_ref[0, 0]
    t = dinv * agg_ref[...] + (dinv * dinv) * h_ref[...] + b_ref[...]
    v = jnp.where(t >= 0, t, a * t)
    vs_ref[...] = v
    cs = jnp.broadcast_to(jnp.sum(v, axis=0, keepdims=True), (8, _D))
    cq = jnp.broadcast_to(jnp.sum(v * v, axis=0, keepdims=True), (8, _D))
    j = pl.program_id(0)

    @pl.when(j % _NBV == 0)
    def _init():
        s1_ref[...] = cs
        s2_ref[...] = cq

    @pl.when(j % _NBV != 0)
    def _accum():
        s1_ref[...] = s1_ref[...] + cs
        s2_ref[...] = s2_ref[...] + cq


def _final_layer(deg, agg, h, b, a):
    return pl.pallas_call(
        _final_layer_body,
        grid=(_NB,),
        in_specs=[
            pl.BlockSpec((_RB, 8), lambda j: (j, 0)),
            pl.BlockSpec((_RB, _D), lambda j: (j, 0)),
            pl.BlockSpec((_RB, _D), lambda j: (j, 0)),
            pl.BlockSpec((1, _D), lambda j: (0, 0)),
            pl.BlockSpec((1, 1), lambda j: (0, 0)),
        ],
        out_specs=[
            pl.BlockSpec((_RB, _D), lambda j: (j, 0)),
            pl.BlockSpec((8, _D), lambda j: (j // _NBV, 0)),
            pl.BlockSpec((8, _D), lambda j: (j // _NBV, 0)),
        ],
        out_shape=[
            jax.ShapeDtypeStruct((2 * _N, _D), jnp.float32),
            jax.ShapeDtypeStruct((16, _D), jnp.float32),
            jax.ShapeDtypeStruct((16, _D), jnp.float32),
        ],
    )(deg, agg, h, b, a)


def _pred1_body(vs_ref, s1_ref, s2_ref, g_ref, bb_ref, w_ref, lb_ref, a_ref,
                z_ref, t1_ref, t2_ref):
    inv_n = 1.0 / _N
    mu = s1_ref[0:1, :] * inv_n
    var = s2_ref[0:1, :] * inv_n - mu * mu
    xn = (vs_ref[...] - mu) * lax.rsqrt(var + _EPS_BN) * g_ref[...] + bb_ref[...]
    z0 = jnp.dot(xn, w_ref[...], preferred_element_type=jnp.float32) + lb_ref[...]
    a = a_ref[0, 0]
    z = jnp.where(z0 >= 0, z0, a * z0)
    z_ref[...] = z
    cs = jnp.broadcast_to(jnp.sum(z, axis=0, keepdims=True), (8, _PH))
    cq = jnp.broadcast_to(jnp.sum(z * z, axis=0, keepdims=True), (8, _PH))
    j = pl.program_id(0)

    @pl.when(j % _NBV == 0)
    def _init():
        t1_ref[...] = cs
        t2_ref[...] = cq

    @pl.when(j % _NBV != 0)
    def _accum():
        t1_ref[...] = t1_ref[...] + cs
        t2_ref[...] = t2_ref[...] + cq


def _pred1(vs, s1, s2, g, bb, w, lb, a):
    return pl.pallas_call(
        _pred1_body,
        grid=(_NB,),
        in_specs=[
            pl.BlockSpec((_RB, _D), lambda j: (j, 0)),
            pl.BlockSpec((8, _D), lambda j: (j // _NBV, 0)),
            pl.BlockSpec((8, _D), lambda j: (j // _NBV, 0)),
            pl.BlockSpec((1, _D), lambda j: (0, 0)),
            pl.BlockSpec((1, _D), lambda j: (0, 0)),
            pl.BlockSpec((_D, _PH), lambda j: (0, 0)),
            pl.BlockSpec((1, _PH), lambda j: (0, 0)),
            pl.BlockSpec((1, 1), lambda j: (0, 0)),
        ],
        out_specs=[
            pl.BlockSpec((_RB, _PH), lambda j: (j, 0)),
            pl.BlockSpec((8, _PH), lambda j: (j // _NBV, 0)),
            pl.BlockSpec((8, _PH), lambda j: (j // _NBV, 0)),
        ],
        out_shape=[
            jax.ShapeDtypeStruct((2 * _N, _PH), jnp.float32),
            jax.ShapeDtypeStruct((16, _PH), jnp.float32),
            jax.ShapeDtypeStruct((16, _PH), jnp.float32),
        ],
    )(vs, s1, s2, g, bb, w, lb, a)


def _loss_body(z1_ref, z2_ref, t1_ref, t2_ref, g_ref, bb_ref, w_ref, lb_ref,
               vs1_ref, vs2_ref, loss_ref):
    inv_n = 1.0 / _N

    def bn(z, k):
        mu = t1_ref[8 * k:8 * k + 1, :] * inv_n
        var = t2_ref[8 * k:8 * k + 1, :] * inv_n - mu * mu
        return (z - mu) * lax.rsqrt(var + _EPS_BN) * g_ref[...] + bb_ref[...]

    p1 = jnp.dot(bn(z1_ref[...], 0), w_ref[...],
                 preferred_element_type=jnp.float32) + lb_ref[...]
    p2 = jnp.dot(bn(z2_ref[...], 1), w_ref[...],
                 preferred_element_type=jnp.float32) + lb_ref[...]

    def nrm(x):
        n = jnp.sqrt(jnp.sum(x * x, axis=1, keepdims=True))
        return x / jnp.maximum(n, 1e-12)

    c1 = jnp.sum(nrm(p1) * nrm(vs2_ref[...]), axis=1)
    c2 = jnp.sum(nrm(p2) * nrm(vs1_ref[...]), axis=1)
    part = jnp.reshape(jnp.sum(4.0 - 2.0 * c1 - 2.0 * c2) * inv_n, (1, 1))
    j = pl.program_id(0)

    @pl.when(j == 0)
    def _init():
        loss_ref[...] = part

    @pl.when(j != 0)
    def _accum():
        loss_ref[...] = loss_ref[...] + part


def _loss(z, t1, t2, g, bb, w, lb, vs):
    return pl.pallas_call(
        _loss_body,
        grid=(_NBV,),
        in_specs=[
            pl.BlockSpec((_RB, _PH), lambda j: (j, 0)),
            pl.BlockSpec((_RB, _PH), lambda j: (j + _NBV, 0)),
            pl.BlockSpec((16, _PH), lambda j: (0, 0)),
            pl.BlockSpec((16, _PH), lambda j: (0, 0)),
            pl.BlockSpec((1, _PH), lambda j: (0, 0)),
            pl.BlockSpec((1, _PH), lambda j: (0, 0)),
            pl.BlockSpec((_PH, _D), lambda j: (0, 0)),
            pl.BlockSpec((1, _D), lambda j: (0, 0)),
            pl.BlockSpec((_RB, _D), lambda j: (j, 0)),
            pl.BlockSpec((_RB, _D), lambda j: (j + _NBV, 0)),
        ],
        out_specs=pl.BlockSpec((1, 1), lambda j: (0, 0)),
        out_shape=jax.ShapeDtypeStruct((1, 1), jnp.float32),
    )(z, z, t1, t2, g, bb, w, lb, vs, vs)


def kernel(x1, x2, edge_index_v1, edge_index_v2, W1, b1, W2, b2, prelu_a,
           bn1_g, bn1_b, lin1_W, lin1_b, prelu2_a, bn2_g, bn2_b, lin2_W,
           lin2_b):
    f32 = jnp.float32

    def prep_idx(col, fill):
        a = col.astype(jnp.int32).reshape(_NS, _EPT)
        pad = jnp.full((_NS, _PAD), fill, jnp.int32)
        return jnp.concatenate([a, pad], axis=1).reshape(_NS, _CPT, _CH)

    src = jnp.stack([
        prep_idx(edge_index_v1[0], 0),
        prep_idx(edge_index_v2[0].astype(jnp.int32) + _N, 0),
    ])
    dst = jnp.stack([
        prep_idx(edge_index_v1[1], _N),
        prep_idx(edge_index_v2[1], _N),
    ])
    ones8 = jnp.ones((_SPLIT, 8), f32)
    zeros = jnp.zeros((_SPLIT, _D), f32)
    xcat = jnp.concatenate([x1, x2], axis=0)

    a1 = jnp.reshape(prelu_a.astype(f32), (1, 1))
    a2 = jnp.reshape(prelu2_a.astype(f32), (1, 1))
    b1r = jnp.reshape(b1, (1, _D))
    b2r = jnp.reshape(b2, (1, _D))
    g1r = jnp.reshape(bn1_g, (1, _D))
    bb1r = jnp.reshape(bn1_b, (1, _D))
    lb1r = jnp.reshape(lin1_b, (1, _PH))
    g2r = jnp.reshape(bn2_g, (1, _PH))
    bb2r = jnp.reshape(bn2_b, (1, _PH))
    lb2r = jnp.reshape(lin2_b, (1, _D))

    deg = _degree_hist(dst, ones8).reshape(2 * _N, 8)

    h1, hp1 = _mm_scale(deg, xcat, W1)
    agg1 = _aggregate(hp1, src, dst, zeros)
    h2, hp2 = _layer_mm(deg, agg1, h1, b1r, a1, W2)
    agg2 = _aggregate(hp2, src, dst, zeros)
    vs, s1, s2 = _final_layer(deg, agg2, h2, b2r, a1)

    z, t1, t2 = _pred1(vs, s1, s2, g1r, bb1r, lin1_W, lb1r, a2)
    lossm = _loss(z, t1, t2, g2r, bb2r, lin2_W, lb2r, vs)

    return vs[:_N], vs[_N:], lossm[0, 0]
